# Initial kernel scaffold; baseline (speedup 1.0000x reference)
import jax
import jax.numpy as jnp
from jax.experimental import pallas as pl


def kernel(x, edge_index, edge_type, conv0_W, conv0_root, conv0_b, conv1_W, conv1_root, conv1_b, conv2_W, conv2_root, conv2_b, bn0_gamma, bn0_beta, bn1_gamma, bn1_beta, cls_W1, cls_b1, cls_W2, cls_b2):
    raise NotImplementedError("write your pallas kernel here")


# speedup vs baseline: 12.5594x; 12.5594x over previous
import functools

import numpy as np
import jax
import jax.numpy as jnp
from jax import lax
from jax.experimental import pallas as pl
from jax.experimental.pallas import tpu as pltpu
from jax.experimental.pallas import tpu_sc as plsc

_N = 10000
_E = 320000
_D = 128
_R = 4
_C = 16
_SEGS_PAD = 40960
_DUMMY = _SEGS_PAD - 1
_NT = 16
_NSC = 2
_BATCH = 128
_NB = 160
_EP = _NT * _NB * _BATCH
_RPT = _SEGS_PAD // _NT
_NCHUNK = 4
_CW = _D // _NCHUNK

_mesh = plsc.VectorSubcoreMesh(core_axis_name="c", subcore_axis_name="s",
                               num_cores=_NSC, num_subcores=_NT)


def _agg_body(h4, src_t, seg_t, out, acc, idx_v, seg_v, rows0, rows1,
              sem0, sem1):
    core = lax.axis_index("c")
    sid = lax.axis_index("s")
    pltpu.sync_copy(src_t.at[sid], idx_v)
    pltpu.sync_copy(seg_t.at[sid], seg_v)

    base = core * 2

    def ibody(t, carry):
        j = t // 8
        k = t % 8
        s = idx_v[j, pl.ds(k * 16, 16)]
        idx_v[j, pl.ds(k * 16, 16)] = s * _NCHUNK + base
        return carry

    lax.fori_loop(0, _NB * 8, ibody, 0)
    row0 = sid * _RPT
    zero16 = jnp.zeros((16,), jnp.float32)
    one16i = jnp.full((16,), 1, jnp.int32)

    for cc in range(2):
        if cc == 1:
            def ibump(t, carry):
                j = t // 8
                k = t % 8
                idx_v[j, pl.ds(k * 16, 16)] = idx_v[j, pl.ds(k * 16, 16)] + one16i
                return carry

            lax.fori_loop(0, _NB * 8, ibump, 0)

        def zrow(t, carry):
            rows0[t // 2, pl.ds((t % 2) * 16, 16)] = zero16
            return carry

        lax.fori_loop(0, _BATCH * 2, zrow, 0)

        def zcp(z, carry):
            pltpu.sync_copy(rows0, acc.at[pl.ds(row0 + z * _BATCH, _BATCH)])
            return carry

        lax.fori_loop(0, _RPT // _BATCH, zcp, 0)
        plsc.subcore_barrier()

        pltpu.async_copy(h4.at[idx_v.at[0]], rows0, sem0)
        pltpu.async_copy(h4.at[idx_v.at[1]], rows1, sem1)

        def pbody(t, carry):
            j = 2 * t
            pltpu.make_async_copy(h4.at[pl.ds(0, _BATCH)], rows0, sem0).wait()
            pltpu.sync_copy(rows0, acc.at[seg_v.at[j]], add=True)
            pltpu.async_copy(h4.at[idx_v.at[j + 2]], rows0, sem0)
            pltpu.make_async_copy(h4.at[pl.ds(0, _BATCH)], rows1, sem1).wait()
            pltpu.sync_copy(rows1, acc.at[seg_v.at[j + 1]], add=True)
            pltpu.async_copy(h4.at[idx_v.at[j + 3]], rows1, sem1)
            return carry

        lax.fori_loop(0, _NB // 2 - 1, pbody, 0)
        pltpu.make_async_copy(h4.at[pl.ds(0, _BATCH)], rows0, sem0).wait()
        pltpu.sync_copy(rows0, acc.at[seg_v.at[_NB - 2]], add=True)
        pltpu.make_async_copy(h4.at[pl.ds(0, _BATCH)], rows1, sem1).wait()
        pltpu.sync_copy(rows1, acc.at[seg_v.at[_NB - 1]], add=True)
        plsc.subcore_barrier()

        off = (base + cc) * _SEGS_PAD + row0
        pltpu.sync_copy(acc.at[pl.ds(row0, _RPT)], out.at[pl.ds(off, _RPT)])


_aggregate = functools.partial(
    pl.kernel,
    out_type=jax.ShapeDtypeStruct((_NCHUNK * _SEGS_PAD, _CW), jnp.float32),
    mesh=_mesh,
    compiler_params=pltpu.CompilerParams(use_tc_tiling_on_sc=False),
    scratch_types=[
        pltpu.VMEM_SHARED((_SEGS_PAD, _CW), jnp.float32),
        pltpu.VMEM((_NB, _BATCH), jnp.int32),
        pltpu.VMEM((_NB, _BATCH), jnp.int32),
        pltpu.VMEM((_BATCH, _CW), jnp.float32),
        pltpu.VMEM((_BATCH, _CW), jnp.float32),
        pltpu.SemaphoreType.DMA,
        pltpu.SemaphoreType.DMA,
    ],
)(_agg_body)


def _cnt_body(seg_t, out, cnt, seg_v, ones_v):
    core = lax.axis_index("c")
    sid = lax.axis_index("s")

    @pl.when(core == 0)
    def _work():
        pltpu.sync_copy(seg_t.at[sid], seg_v)
        zero16 = jnp.zeros((16,), jnp.float32)

        def zrow(r, carry):
            ones_v[r, pl.ds(0, 16)] = zero16
            return carry

        lax.fori_loop(0, _BATCH, zrow, 0)
        row0 = sid * _RPT

        def zcp(z, carry):
            pltpu.sync_copy(ones_v, cnt.at[pl.ds(row0 + z * _BATCH, _BATCH)])
            return carry

        lax.fori_loop(0, _RPT // _BATCH, zcp, 0)
        one16 = jnp.ones((16,), jnp.float32)

        def orow(r, carry):
            ones_v[r, pl.ds(0, 16)] = one16
            return carry

        lax.fori_loop(0, _BATCH, orow, 0)
        plsc.subcore_barrier()

        def body(j, carry):
            pltpu.sync_copy(ones_v, cnt.at[seg_v.at[j]], add=True)
            return carry

        lax.fori_loop(0, _NB, body, 0)
        plsc.subcore_barrier()
        pltpu.sync_copy(cnt.at[pl.ds(row0, _RPT)], out.at[pl.ds(row0, _RPT)])


_counts = functools.partial(
    pl.kernel,
    out_type=jax.ShapeDtypeStruct((_SEGS_PAD, 16), jnp.float32),
    mesh=_mesh,
    compiler_params=pltpu.CompilerParams(use_tc_tiling_on_sc=False),
    scratch_types=[
        pltpu.VMEM_SHARED((_SEGS_PAD, 16), jnp.float32),
        pltpu.VMEM((_NB, _BATCH), jnp.int32),
        pltpu.VMEM((_BATCH, 16), jnp.float32),
    ],
)(_cnt_body)


_B = 1000
_NBLK = _N // _B
_BN_SCALE = np.float32(1.0 / np.sqrt(1.0 + 1e-5))


def _scaled_agg(acc_ref, cnt_ref, h_ref, w4_ref, wr_ref, b_ref, resid):
    lane = lax.broadcasted_iota(jnp.int32, (_R, _D), 1)
    row = lax.broadcasted_iota(jnp.int32, (_R, _D), 0)
    expand = (lane // _CW == row).astype(jnp.float32)
    inv = 1.0 / jnp.maximum(cnt_ref[...], 1.0)
    invrep = jnp.dot(inv, expand, preferred_element_type=jnp.float32)
    h = h_ref[...]
    t = jnp.dot(h, wr_ref[...], preferred_element_type=jnp.float32) + b_ref[...]
    for c in range(_NCHUNK):
        t = t + jnp.dot(acc_ref[c] * invrep, w4_ref[c],
                        preferred_element_type=jnp.float32)
    if resid:
        t = t + h
    return t


def _tc_layer_body(acc_ref, cnt_ref, h_ref, w4_ref, wr_ref, b_ref, g_ref,
                   bt_ref, o_ref, *, resid):
    t = _scaled_agg(acc_ref, cnt_ref, h_ref, w4_ref, wr_ref, b_ref, resid)
    t = g_ref[...] * t * _BN_SCALE + bt_ref[...]
    o_ref[...] = jnp.where(t > 0, t, jnp.exp(t) - 1.0)


def _tc_final_body(acc_ref, cnt_ref, h_ref, w4_ref, wr_ref, b_ref, w1_ref,
                   b1_ref, w2_ref, b2_ref, o_ref):
    t = _scaled_agg(acc_ref, cnt_ref, h_ref, w4_ref, wr_ref, b_ref, True)
    z1 = jnp.dot(t, w1_ref[...], preferred_element_type=jnp.float32) + b1_ref[...]
    z1 = jnp.where(z1 > 0, z1, jnp.exp(z1) - 1.0)
    z = jnp.dot(z1, w2_ref[...], preferred_element_type=jnp.float32) + b2_ref[...]
    m = jnp.max(z, axis=1, keepdims=True)
    lse = m + jnp.log(jnp.sum(jnp.exp(z - m), axis=1, keepdims=True))
    o_ref[...] = z - lse


_common_specs = [
    pl.BlockSpec((_NCHUNK, _B, _D), lambda i: (0, i, 0)),
    pl.BlockSpec((_B, _R), lambda i: (i, 0)),
    pl.BlockSpec((_B, _D), lambda i: (i, 0)),
    pl.BlockSpec((_NCHUNK, _D, _D), lambda i: (0, 0, 0)),
    pl.BlockSpec((_D, _D), lambda i: (0, 0)),
    pl.BlockSpec((1, _D), lambda i: (0, 0)),
]


def _tc_layer(acc4, cnt4, h, w4, wroot, b, gamma, beta, resid):
    return pl.pallas_call(
        functools.partial(_tc_layer_body, resid=resid),
        grid=(_NBLK,),
        in_specs=_common_specs + [
            pl.BlockSpec((1, _D), lambda i: (0, 0)),
            pl.BlockSpec((1, _D), lambda i: (0, 0)),
        ],
        out_specs=pl.BlockSpec((_B, _D), lambda i: (i, 0)),
        out_shape=jax.ShapeDtypeStruct((_N, _D), jnp.float32),
    )(acc4, cnt4, h, w4, wroot, b, gamma, beta)


def _tc_final(acc4, cnt4, h, w4, wroot, b, w1, b1, w2, b2):
    return pl.pallas_call(
        _tc_final_body,
        grid=(_NBLK,),
        in_specs=_common_specs + [
            pl.BlockSpec((_D, _D // 2), lambda i: (0, 0)),
            pl.BlockSpec((1, _D // 2), lambda i: (0, 0)),
            pl.BlockSpec((_D // 2, _C), lambda i: (0, 0)),
            pl.BlockSpec((1, _C), lambda i: (0, 0)),
        ],
        out_specs=pl.BlockSpec((_B, _C), lambda i: (i, 0)),
        out_shape=jax.ShapeDtypeStruct((_N, _C), jnp.float32),
    )(acc4, cnt4, h, w4, wroot, b, w1, b1, w2, b2)


def _reorg_w(w):
    return jnp.stack([w[:, c * _CW:(c + 1) * _CW, :].reshape(_R * _CW, _D)
                      for c in range(_NCHUNK)])


def kernel(x, edge_index, edge_type,
           conv0_W, conv0_root, conv0_b,
           conv1_W, conv1_root, conv1_b,
           conv2_W, conv2_root, conv2_b,
           bn0_gamma, bn0_beta, bn1_gamma, bn1_beta,
           cls_W1, cls_b1, cls_W2, cls_b2):
    src = edge_index[0].astype(jnp.int32)
    seg = edge_index[1].astype(jnp.int32) * _R + edge_type.astype(jnp.int32)
    pad = _EP - _E
    src_t = jnp.concatenate([src, jnp.zeros((pad,), jnp.int32)]
                            ).reshape(_NT, _NB, _BATCH)
    seg_t = jnp.concatenate([seg, jnp.full((pad,), _DUMMY, jnp.int32)]
                            ).reshape(_NT, _NB, _BATCH)

    cnt4 = _counts(seg_t)[:, 0].reshape(_SEGS_PAD // _R, _R)

    def agg4(h):
        acc = _aggregate(h.reshape(_N * _NCHUNK, _CW), src_t, seg_t)
        return acc.reshape(_NCHUNK, _SEGS_PAD // _R, _D)

    h1 = _tc_layer(agg4(x), cnt4, x, _reorg_w(conv0_W), conv0_root,
                   conv0_b.reshape(1, _D), bn0_gamma.reshape(1, _D),
                   bn0_beta.reshape(1, _D), resid=False)
    h2 = _tc_layer(agg4(h1), cnt4, h1, _reorg_w(conv1_W), conv1_root,
                   conv1_b.reshape(1, _D), bn1_gamma.reshape(1, _D),
                   bn1_beta.reshape(1, _D), resid=True)
    return _tc_final(agg4(h2), cnt4, h2, _reorg_w(conv2_W), conv2_root,
                     conv2_b.reshape(1, _D), cls_W1,
                     cls_b1.reshape(1, _D // 2), cls_W2,
                     cls_b2.reshape(1, _C))


# speedup vs baseline: 12.7972x; 1.0189x over previous
import functools

import numpy as np
import jax
import jax.numpy as jnp
from jax import lax
from jax.experimental import pallas as pl
from jax.experimental.pallas import tpu as pltpu
from jax.experimental.pallas import tpu_sc as plsc

_N = 10000
_E = 320000
_D = 128
_R = 4
_C = 16
_SEGS_PAD = 40960
_DUMMY = _SEGS_PAD - 1
_NT = 16
_NSC = 2
_BATCH = 128
_NB = 160
_EP = _NT * _NB * _BATCH
_RPT = _SEGS_PAD // _NT
_NCHUNK = 4
_CW = _D // _NCHUNK

_mesh = plsc.VectorSubcoreMesh(core_axis_name="c", subcore_axis_name="s",
                               num_cores=_NSC, num_subcores=_NT)


_HB = _NB // 2


def _agg_body(h4, src_t, seg_t, out, acc, idx_v, seg_v, rows0, rows1,
              rows2, rows3, gsem0, gsem1, gsem2, gsem3,
              ssem0, ssem1, ssem2, ssem3):
    core = lax.axis_index("c")
    sid = lax.axis_index("s")
    base = core * 2
    row0 = sid * _RPT
    zero16 = jnp.zeros((16,), jnp.float32)
    bufs = (rows0, rows1, rows2, rows3)
    gsems = (gsem0, gsem1, gsem2, gsem3)
    ssems = (ssem0, ssem1, ssem2, ssem3)

    def startg(j, rows, sem):
        pltpu.async_copy(h4.at[idx_v.at[j]], rows, sem)

    def waitg(rows, sem):
        pltpu.make_async_copy(h4.at[pl.ds(0, _BATCH)], rows, sem).wait()

    def starts(j, rows, sem):
        pltpu.async_copy(rows, acc.at[seg_v.at[j]], sem, add=True)

    def waits(rows, sem):
        pltpu.make_async_copy(rows, acc.at[pl.ds(0, _BATCH)], sem).wait()

    for cc in range(2):
        chunk = base + cc
        def zrow(t, carry):
            rows0[t // 2, pl.ds((t % 2) * 16, 16)] = zero16
            return carry

        lax.fori_loop(0, _BATCH * 2, zrow, 0)

        def zcp(z, carry):
            pltpu.sync_copy(rows0, acc.at[pl.ds(row0 + z * _BATCH, _BATCH)])
            return carry

        lax.fori_loop(0, _RPT // _BATCH, zcp, 0)
        plsc.subcore_barrier()

        for half in range(2):
            pltpu.sync_copy(src_t.at[sid, pl.ds(half * _HB, _HB)], idx_v)
            pltpu.sync_copy(seg_t.at[sid, pl.ds(half * _HB, _HB)], seg_v)

            def ibody(t, carry):
                j = t // 8
                k = t % 8
                sv = idx_v[j, pl.ds(k * 16, 16)]
                idx_v[j, pl.ds(k * 16, 16)] = sv * _NCHUNK + chunk
                return carry

            lax.fori_loop(0, _HB * 8, ibody, 0)

            startg(0, bufs[0], gsems[0])
            startg(1, bufs[1], gsems[1])
            for j in range(2):
                waitg(bufs[j], gsems[j])
                starts(j, bufs[j], ssems[j])
                startg(j + 2, bufs[j + 2], gsems[j + 2])

            def pbody(t, carry):
                j0 = 4 * t + 2
                for k in range(4):
                    b = (2 + k) % 4
                    nb = k % 4
                    waitg(bufs[b], gsems[b])
                    starts(j0 + k, bufs[b], ssems[b])
                    waits(bufs[nb], ssems[nb])
                    startg(j0 + k + 2, bufs[nb], gsems[nb])
                return carry

            lax.fori_loop(0, (_HB - 4) // 4, pbody, 0)
            for j in range(_HB - 2, _HB):
                b = j % 4
                waitg(bufs[b], gsems[b])
                starts(j, bufs[b], ssems[b])
            for b in range(4):
                waits(bufs[b], ssems[b])
        plsc.subcore_barrier()

        off = chunk * _SEGS_PAD + row0
        pltpu.sync_copy(acc.at[pl.ds(row0, _RPT)], out.at[pl.ds(off, _RPT)])


_aggregate = functools.partial(
    pl.kernel,
    out_type=jax.ShapeDtypeStruct((_NCHUNK * _SEGS_PAD, _CW), jnp.float32),
    mesh=_mesh,
    compiler_params=pltpu.CompilerParams(use_tc_tiling_on_sc=False),
    scratch_types=[
        pltpu.VMEM_SHARED((_SEGS_PAD, _CW), jnp.float32),
        pltpu.VMEM((_HB, _BATCH), jnp.int32),
        pltpu.VMEM((_HB, _BATCH), jnp.int32),
        pltpu.VMEM((_BATCH, _CW), jnp.float32),
        pltpu.VMEM((_BATCH, _CW), jnp.float32),
        pltpu.VMEM((_BATCH, _CW), jnp.float32),
        pltpu.VMEM((_BATCH, _CW), jnp.float32),
        pltpu.SemaphoreType.DMA,
        pltpu.SemaphoreType.DMA,
        pltpu.SemaphoreType.DMA,
        pltpu.SemaphoreType.DMA,
        pltpu.SemaphoreType.DMA,
        pltpu.SemaphoreType.DMA,
        pltpu.SemaphoreType.DMA,
        pltpu.SemaphoreType.DMA,
    ],
)(_agg_body)


def _cnt_body(seg_t, out, cnt, seg_v, ones_v):
    core = lax.axis_index("c")
    sid = lax.axis_index("s")

    @pl.when(core == 0)
    def _work():
        pltpu.sync_copy(seg_t.at[sid], seg_v)
        zero16 = jnp.zeros((16,), jnp.float32)

        def zrow(r, carry):
            ones_v[r, pl.ds(0, 16)] = zero16
            return carry

        lax.fori_loop(0, _BATCH, zrow, 0)
        row0 = sid * _RPT

        def zcp(z, carry):
            pltpu.sync_copy(ones_v, cnt.at[pl.ds(row0 + z * _BATCH, _BATCH)])
            return carry

        lax.fori_loop(0, _RPT // _BATCH, zcp, 0)
        one16 = jnp.ones((16,), jnp.float32)

        def orow(r, carry):
            ones_v[r, pl.ds(0, 16)] = one16
            return carry

        lax.fori_loop(0, _BATCH, orow, 0)
        plsc.subcore_barrier()

        def body(j, carry):
            pltpu.sync_copy(ones_v, cnt.at[seg_v.at[j]], add=True)
            return carry

        lax.fori_loop(0, _NB, body, 0)
        plsc.subcore_barrier()
        pltpu.sync_copy(cnt.at[pl.ds(row0, _RPT)], out.at[pl.ds(row0, _RPT)])


_counts = functools.partial(
    pl.kernel,
    out_type=jax.ShapeDtypeStruct((_SEGS_PAD, 16), jnp.float32),
    mesh=_mesh,
    compiler_params=pltpu.CompilerParams(use_tc_tiling_on_sc=False),
    scratch_types=[
        pltpu.VMEM_SHARED((_SEGS_PAD, 16), jnp.float32),
        pltpu.VMEM((_NB, _BATCH), jnp.int32),
        pltpu.VMEM((_BATCH, 16), jnp.float32),
    ],
)(_cnt_body)


_B = 1000
_NBLK = _N // _B
_BN_SCALE = np.float32(1.0 / np.sqrt(1.0 + 1e-5))


def _scaled_agg(acc_ref, cnt_ref, h_ref, w4_ref, wr_ref, b_ref, resid):
    lane = lax.broadcasted_iota(jnp.int32, (_R, _D), 1)
    row = lax.broadcasted_iota(jnp.int32, (_R, _D), 0)
    expand = (lane // _CW == row).astype(jnp.float32)
    inv = 1.0 / jnp.maximum(cnt_ref[...], 1.0)
    invrep = jnp.dot(inv, expand, preferred_element_type=jnp.float32)
    h = h_ref[...]
    t = jnp.dot(h, wr_ref[...], preferred_element_type=jnp.float32) + b_ref[...]
    for c in range(_NCHUNK):
        t = t + jnp.dot(acc_ref[c] * invrep, w4_ref[c],
                        preferred_element_type=jnp.float32)
    if resid:
        t = t + h
    return t


def _tc_layer_body(acc_ref, cnt_ref, h_ref, w4_ref, wr_ref, b_ref, g_ref,
                   bt_ref, o_ref, *, resid):
    t = _scaled_agg(acc_ref, cnt_ref, h_ref, w4_ref, wr_ref, b_ref, resid)
    t = g_ref[...] * t * _BN_SCALE + bt_ref[...]
    o_ref[...] = jnp.where(t > 0, t, jnp.exp(t) - 1.0)


def _tc_final_body(acc_ref, cnt_ref, h_ref, w4_ref, wr_ref, b_ref, w1_ref,
                   b1_ref, w2_ref, b2_ref, o_ref):
    t = _scaled_agg(acc_ref, cnt_ref, h_ref, w4_ref, wr_ref, b_ref, True)
    z1 = jnp.dot(t, w1_ref[...], preferred_element_type=jnp.float32) + b1_ref[...]
    z1 = jnp.where(z1 > 0, z1, jnp.exp(z1) - 1.0)
    z = jnp.dot(z1, w2_ref[...], preferred_element_type=jnp.float32) + b2_ref[...]
    m = jnp.max(z, axis=1, keepdims=True)
    lse = m + jnp.log(jnp.sum(jnp.exp(z - m), axis=1, keepdims=True))
    o_ref[...] = z - lse


_common_specs = [
    pl.BlockSpec((_NCHUNK, _B, _D), lambda i: (0, i, 0)),
    pl.BlockSpec((_B, _R), lambda i: (i, 0)),
    pl.BlockSpec((_B, _D), lambda i: (i, 0)),
    pl.BlockSpec((_NCHUNK, _D, _D), lambda i: (0, 0, 0)),
    pl.BlockSpec((_D, _D), lambda i: (0, 0)),
    pl.BlockSpec((1, _D), lambda i: (0, 0)),
]


def _tc_layer(acc4, cnt4, h, w4, wroot, b, gamma, beta, resid):
    return pl.pallas_call(
        functools.partial(_tc_layer_body, resid=resid),
        grid=(_NBLK,),
        in_specs=_common_specs + [
            pl.BlockSpec((1, _D), lambda i: (0, 0)),
            pl.BlockSpec((1, _D), lambda i: (0, 0)),
        ],
        out_specs=pl.BlockSpec((_B, _D), lambda i: (i, 0)),
        out_shape=jax.ShapeDtypeStruct((_N, _D), jnp.float32),
    )(acc4, cnt4, h, w4, wroot, b, gamma, beta)


def _tc_final(acc4, cnt4, h, w4, wroot, b, w1, b1, w2, b2):
    return pl.pallas_call(
        _tc_final_body,
        grid=(_NBLK,),
        in_specs=_common_specs + [
            pl.BlockSpec((_D, _D // 2), lambda i: (0, 0)),
            pl.BlockSpec((1, _D // 2), lambda i: (0, 0)),
            pl.BlockSpec((_D // 2, _C), lambda i: (0, 0)),
            pl.BlockSpec((1, _C), lambda i: (0, 0)),
        ],
        out_specs=pl.BlockSpec((_B, _C), lambda i: (i, 0)),
        out_shape=jax.ShapeDtypeStruct((_N, _C), jnp.float32),
    )(acc4, cnt4, h, w4, wroot, b, w1, b1, w2, b2)


def _reorg_w(w):
    return jnp.stack([w[:, c * _CW:(c + 1) * _CW, :].reshape(_R * _CW, _D)
                      for c in range(_NCHUNK)])


def kernel(x, edge_index, edge_type,
           conv0_W, conv0_root, conv0_b,
           conv1_W, conv1_root, conv1_b,
           conv2_W, conv2_root, conv2_b,
           bn0_gamma, bn0_beta, bn1_gamma, bn1_beta,
           cls_W1, cls_b1, cls_W2, cls_b2):
    src = edge_index[0].astype(jnp.int32)
    seg = edge_index[1].astype(jnp.int32) * _R + edge_type.astype(jnp.int32)
    pad = _EP - _E
    src_t = jnp.concatenate([src, jnp.zeros((pad,), jnp.int32)]
                            ).reshape(_NT, _NB, _BATCH)
    seg_t = jnp.concatenate([seg, jnp.full((pad,), _DUMMY, jnp.int32)]
                            ).reshape(_NT, _NB, _BATCH)

    cnt4 = _counts(seg_t)[:, 0].reshape(_SEGS_PAD // _R, _R)

    def agg4(h):
        acc = _aggregate(h.reshape(_N * _NCHUNK, _CW), src_t, seg_t)
        return acc.reshape(_NCHUNK, _SEGS_PAD // _R, _D)

    h1 = _tc_layer(agg4(x), cnt4, x, _reorg_w(conv0_W), conv0_root,
                   conv0_b.reshape(1, _D), bn0_gamma.reshape(1, _D),
                   bn0_beta.reshape(1, _D), resid=False)
    h2 = _tc_layer(agg4(h1), cnt4, h1, _reorg_w(conv1_W), conv1_root,
                   conv1_b.reshape(1, _D), bn1_gamma.reshape(1, _D),
                   bn1_beta.reshape(1, _D), resid=True)
    return _tc_final(agg4(h2), cnt4, h2, _reorg_w(conv2_W), conv2_root,
                     conv2_b.reshape(1, _D), cls_W1,
                     cls_b1.reshape(1, _D // 2), cls_W2,
                     cls_b2.reshape(1, _C))


# speedup vs baseline: 12.8445x; 1.0037x over previous
import functools

import numpy as np
import jax
import jax.numpy as jnp
from jax import lax
from jax.experimental import pallas as pl
from jax.experimental.pallas import tpu as pltpu
from jax.experimental.pallas import tpu_sc as plsc

_N = 10000
_E = 320000
_D = 128
_R = 4
_C = 16
_SEGS_PAD = 40960
_DUMMY = _SEGS_PAD - 1
_NT = 16
_NSC = 2
_BATCH = 128
_NB = 160
_EP = _NT * _NB * _BATCH
_RPT = _SEGS_PAD // _NT
_NCHUNK = 4
_CW = _D // _NCHUNK

_mesh = plsc.VectorSubcoreMesh(core_axis_name="c", subcore_axis_name="s",
                               num_cores=_NSC, num_subcores=_NT)


_HB = _NB // 2


def _agg_body(h4, src_t, seg_t, cnt_dep, out, acc, idx_v, seg_v, rows0, rows1,
              rows2, rows3, gsem0, gsem1, gsem2, gsem3,
              ssem0, ssem1, ssem2, ssem3):
    del cnt_dep
    core = lax.axis_index("c")
    sid = lax.axis_index("s")
    base = core * 2
    row0 = sid * _RPT
    zero16 = jnp.zeros((16,), jnp.float32)
    bufs = (rows0, rows1, rows2, rows3)
    gsems = (gsem0, gsem1, gsem2, gsem3)
    ssems = (ssem0, ssem1, ssem2, ssem3)

    def startg(j, rows, sem):
        pltpu.async_copy(h4.at[idx_v.at[j]], rows, sem)

    def waitg(rows, sem):
        pltpu.make_async_copy(h4.at[pl.ds(0, _BATCH)], rows, sem).wait()

    def starts(j, rows, sem):
        pltpu.async_copy(rows, acc.at[seg_v.at[j]], sem, add=True)

    def waits(rows, sem):
        pltpu.make_async_copy(rows, acc.at[pl.ds(0, _BATCH)], sem).wait()

    for cc in range(2):
        chunk = base + cc
        def zrow(t, carry):
            rows0[t // 2, pl.ds((t % 2) * 16, 16)] = zero16
            return carry

        lax.fori_loop(0, _BATCH * 2, zrow, 0)

        def zcp(z, carry):
            pltpu.sync_copy(rows0, acc.at[pl.ds(row0 + z * _BATCH, _BATCH)])
            return carry

        lax.fori_loop(0, _RPT // _BATCH, zcp, 0)
        plsc.subcore_barrier()

        for half in range(2):
            pltpu.sync_copy(src_t.at[sid, pl.ds(half * _HB, _HB)], idx_v)
            pltpu.sync_copy(seg_t.at[sid, pl.ds(half * _HB, _HB)], seg_v)

            def ibody(t, carry):
                j = t // 8
                k = t % 8
                sv = idx_v[j, pl.ds(k * 16, 16)]
                idx_v[j, pl.ds(k * 16, 16)] = sv * _NCHUNK + chunk
                return carry

            lax.fori_loop(0, _HB * 8, ibody, 0)

            startg(0, bufs[0], gsems[0])
            startg(1, bufs[1], gsems[1])
            for j in range(2):
                waitg(bufs[j], gsems[j])
                starts(j, bufs[j], ssems[j])
                startg(j + 2, bufs[j + 2], gsems[j + 2])

            def pbody(t, carry):
                j0 = 4 * t + 2
                for k in range(4):
                    b = (2 + k) % 4
                    nb = k % 4
                    waitg(bufs[b], gsems[b])
                    starts(j0 + k, bufs[b], ssems[b])
                    waits(bufs[nb], ssems[nb])
                    startg(j0 + k + 2, bufs[nb], gsems[nb])
                return carry

            lax.fori_loop(0, (_HB - 4) // 4, pbody, 0)
            for j in range(_HB - 2, _HB):
                b = j % 4
                waitg(bufs[b], gsems[b])
                starts(j, bufs[b], ssems[b])
            for b in range(4):
                waits(bufs[b], ssems[b])
        plsc.subcore_barrier()

        off = chunk * _SEGS_PAD + row0
        pltpu.sync_copy(acc.at[pl.ds(row0, _RPT)], out.at[pl.ds(off, _RPT)])


_aggregate = functools.partial(
    pl.kernel,
    out_type=jax.ShapeDtypeStruct((_NCHUNK * _SEGS_PAD, _CW), jnp.float32),
    mesh=_mesh,
    compiler_params=pltpu.CompilerParams(use_tc_tiling_on_sc=False),
    scratch_types=[
        pltpu.VMEM_SHARED((_SEGS_PAD, _CW), jnp.float32),
        pltpu.VMEM((_HB, _BATCH), jnp.int32),
        pltpu.VMEM((_HB, _BATCH), jnp.int32),
        pltpu.VMEM((_BATCH, _CW), jnp.float32),
        pltpu.VMEM((_BATCH, _CW), jnp.float32),
        pltpu.VMEM((_BATCH, _CW), jnp.float32),
        pltpu.VMEM((_BATCH, _CW), jnp.float32),
        pltpu.SemaphoreType.DMA,
        pltpu.SemaphoreType.DMA,
        pltpu.SemaphoreType.DMA,
        pltpu.SemaphoreType.DMA,
        pltpu.SemaphoreType.DMA,
        pltpu.SemaphoreType.DMA,
        pltpu.SemaphoreType.DMA,
        pltpu.SemaphoreType.DMA,
    ],
)(_agg_body)


def _cnt_body(seg_t, out, cnt, seg_v, ones_v):
    core = lax.axis_index("c")
    sid = lax.axis_index("s")

    @pl.when(core == 0)
    def _work():
        pltpu.sync_copy(seg_t.at[sid], seg_v)
        zero16 = jnp.zeros((16,), jnp.float32)

        def zrow(r, carry):
            ones_v[r, pl.ds(0, 16)] = zero16
            return carry

        lax.fori_loop(0, _BATCH, zrow, 0)
        row0 = sid * _RPT

        def zcp(z, carry):
            pltpu.sync_copy(ones_v, cnt.at[pl.ds(row0 + z * _BATCH, _BATCH)])
            return carry

        lax.fori_loop(0, _RPT // _BATCH, zcp, 0)
        one16 = jnp.ones((16,), jnp.float32)

        def orow(r, carry):
            ones_v[r, pl.ds(0, 16)] = one16
            return carry

        lax.fori_loop(0, _BATCH, orow, 0)
        plsc.subcore_barrier()

        def body(j, carry):
            pltpu.sync_copy(ones_v, cnt.at[seg_v.at[j]], add=True)
            return carry

        lax.fori_loop(0, _NB, body, 0)
        plsc.subcore_barrier()
        pltpu.sync_copy(cnt.at[pl.ds(row0, _RPT)], out.at[pl.ds(row0, _RPT)])


_counts = functools.partial(
    pl.kernel,
    out_type=jax.ShapeDtypeStruct((_SEGS_PAD, 16), jnp.float32),
    mesh=_mesh,
    compiler_params=pltpu.CompilerParams(use_tc_tiling_on_sc=False),
    scratch_types=[
        pltpu.VMEM_SHARED((_SEGS_PAD, 16), jnp.float32),
        pltpu.VMEM((_NB, _BATCH), jnp.int32),
        pltpu.VMEM((_BATCH, 16), jnp.float32),
    ],
)(_cnt_body)


_B = 1000
_NBLK = _N // _B
_BN_SCALE = np.float32(1.0 / np.sqrt(1.0 + 1e-5))


def _scaled_agg(acc_ref, cnt_ref, h_ref, w4_ref, wr_ref, b_ref, resid):
    lane = lax.broadcasted_iota(jnp.int32, (_R, _D), 1)
    row = lax.broadcasted_iota(jnp.int32, (_R, _D), 0)
    expand = (lane // _CW == row).astype(jnp.float32)
    inv = 1.0 / jnp.maximum(cnt_ref[...], 1.0)
    invrep = jnp.dot(inv, expand, preferred_element_type=jnp.float32)
    h = h_ref[...]
    t = jnp.dot(h, wr_ref[...], preferred_element_type=jnp.float32) + b_ref[...]
    for c in range(_NCHUNK):
        t = t + jnp.dot(acc_ref[c] * invrep, w4_ref[c],
                        preferred_element_type=jnp.float32)
    if resid:
        t = t + h
    return t


def _tc_layer_body(acc_ref, cnt_ref, h_ref, w4_ref, wr_ref, b_ref, g_ref,
                   bt_ref, o_ref, *, resid):
    t = _scaled_agg(acc_ref, cnt_ref, h_ref, w4_ref, wr_ref, b_ref, resid)
    t = g_ref[...] * t * _BN_SCALE + bt_ref[...]
    o_ref[...] = jnp.where(t > 0, t, jnp.exp(t) - 1.0)


def _tc_final_body(acc_ref, cnt_ref, h_ref, w4_ref, wr_ref, b_ref, w1_ref,
                   b1_ref, w2_ref, b2_ref, o_ref):
    t = _scaled_agg(acc_ref, cnt_ref, h_ref, w4_ref, wr_ref, b_ref, True)
    z1 = jnp.dot(t, w1_ref[...], preferred_element_type=jnp.float32) + b1_ref[...]
    z1 = jnp.where(z1 > 0, z1, jnp.exp(z1) - 1.0)
    z = jnp.dot(z1, w2_ref[...], preferred_element_type=jnp.float32) + b2_ref[...]
    m = jnp.max(z, axis=1, keepdims=True)
    lse = m + jnp.log(jnp.sum(jnp.exp(z - m), axis=1, keepdims=True))
    o_ref[...] = z - lse


_common_specs = [
    pl.BlockSpec((_NCHUNK, _B, _D), lambda i: (0, i, 0)),
    pl.BlockSpec((_B, _R), lambda i: (i, 0)),
    pl.BlockSpec((_B, _D), lambda i: (i, 0)),
    pl.BlockSpec((_NCHUNK, _D, _D), lambda i: (0, 0, 0)),
    pl.BlockSpec((_D, _D), lambda i: (0, 0)),
    pl.BlockSpec((1, _D), lambda i: (0, 0)),
]


def _tc_layer(acc4, cnt4, h, w4, wroot, b, gamma, beta, resid):
    return pl.pallas_call(
        functools.partial(_tc_layer_body, resid=resid),
        grid=(_NBLK,),
        in_specs=_common_specs + [
            pl.BlockSpec((1, _D), lambda i: (0, 0)),
            pl.BlockSpec((1, _D), lambda i: (0, 0)),
        ],
        out_specs=pl.BlockSpec((_B, _D), lambda i: (i, 0)),
        out_shape=jax.ShapeDtypeStruct((_N, _D), jnp.float32),
    )(acc4, cnt4, h, w4, wroot, b, gamma, beta)


def _tc_final(acc4, cnt4, h, w4, wroot, b, w1, b1, w2, b2):
    return pl.pallas_call(
        _tc_final_body,
        grid=(_NBLK,),
        in_specs=_common_specs + [
            pl.BlockSpec((_D, _D // 2), lambda i: (0, 0)),
            pl.BlockSpec((1, _D // 2), lambda i: (0, 0)),
            pl.BlockSpec((_D // 2, _C), lambda i: (0, 0)),
            pl.BlockSpec((1, _C), lambda i: (0, 0)),
        ],
        out_specs=pl.BlockSpec((_B, _C), lambda i: (i, 0)),
        out_shape=jax.ShapeDtypeStruct((_N, _C), jnp.float32),
    )(acc4, cnt4, h, w4, wroot, b, w1, b1, w2, b2)


def _reorg_w(w):
    return jnp.stack([w[:, c * _CW:(c + 1) * _CW, :].reshape(_R * _CW, _D)
                      for c in range(_NCHUNK)])


def kernel(x, edge_index, edge_type,
           conv0_W, conv0_root, conv0_b,
           conv1_W, conv1_root, conv1_b,
           conv2_W, conv2_root, conv2_b,
           bn0_gamma, bn0_beta, bn1_gamma, bn1_beta,
           cls_W1, cls_b1, cls_W2, cls_b2):
    src = edge_index[0].astype(jnp.int32)
    seg = edge_index[1].astype(jnp.int32) * _R + edge_type.astype(jnp.int32)
    pad = _EP - _E
    src_t = jnp.concatenate([src, jnp.zeros((pad,), jnp.int32)]
                            ).reshape(_NT, _NB, _BATCH)
    seg_t = jnp.concatenate([seg, jnp.full((pad,), _DUMMY, jnp.int32)]
                            ).reshape(_NT, _NB, _BATCH)

    cnt2 = _counts(seg_t)
    cnt4 = cnt2[:, 0].reshape(_SEGS_PAD // _R, _R)

    def agg4(h):
        acc = _aggregate(h.reshape(_N * _NCHUNK, _CW), src_t, seg_t, cnt2)
        return acc.reshape(_NCHUNK, _SEGS_PAD // _R, _D)

    h1 = _tc_layer(agg4(x), cnt4, x, _reorg_w(conv0_W), conv0_root,
                   conv0_b.reshape(1, _D), bn0_gamma.reshape(1, _D),
                   bn0_beta.reshape(1, _D), resid=False)
    h2 = _tc_layer(agg4(h1), cnt4, h1, _reorg_w(conv1_W), conv1_root,
                   conv1_b.reshape(1, _D), bn1_gamma.reshape(1, _D),
                   bn1_beta.reshape(1, _D), resid=True)
    return _tc_final(agg4(h2), cnt4, h2, _reorg_w(conv2_W), conv2_root,
                     conv2_b.reshape(1, _D), cls_W1,
                     cls_b1.reshape(1, _D // 2), cls_W2,
                     cls_b2.reshape(1, _C))
